# baseline (speedup 1.0000x reference)
import jax
import jax.numpy as jnp
from jax.experimental import pallas as pl
from jax.experimental.pallas import tpu as pltpu

_T, _B, _OBJ, _FEAT, _R, _EMB, _NBL, _MP = 2, 8, 128, 64, 3, 16, 2, 2
_G = _T * _B
_GPB = 4
_RO = _R * _OBJ
_RE = _R * _EMB


def _gnn_body(blk_ref, unary_ref, We_ref, be_ref, Wr_ref, Wstk_ref, bg_ref,
              Wd_ref, bd_ref, Wb_ref, bb_ref, out_ref):
    f32 = jnp.float32
    i_ro = jax.lax.broadcasted_iota(jnp.int32, (_RO, _RE), 0)
    j_re = jax.lax.broadcasted_iota(jnp.int32, (_RO, _RE), 1)
    M = (i_ro % _R == j_re // _EMB).astype(f32)
    i_s = jax.lax.broadcasted_iota(jnp.int32, (_RO, _OBJ), 0)
    d_s = jax.lax.broadcasted_iota(jnp.int32, (_RO, _OBJ), 1)
    S = (i_s // _R == d_s).astype(f32)

    mask = (blk_ref[...] != 0).astype(f32)
    xs = jnp.dot(unary_ref[...], We_ref[...],
                 preferred_element_type=f32) + be_ref[...]

    Bn = []
    x = []
    for k in range(_GPB):
        mk = mask[k * _OBJ:(k + 1) * _OBJ, :]
        deg = jnp.sum(mk, axis=0, keepdims=True)
        Bn.append(mk * (1.0 / jnp.maximum(deg, 1.0)))
        x.append(xs[k * _OBJ:(k + 1) * _OBJ, :])

    for l in range(_NBL):
        w_root = Wr_ref[l]
        w_stk = Wstk_ref[l]
        b = bg_ref[l:l + 1, :]
        for _ in range(_MP):
            for k in range(_GPB):
                draw = jax.lax.dot_general(
                    Bn[k], x[k], (((0,), (0,)), ((), ())),
                    preferred_element_type=f32)
                full = jnp.dot(draw, w_stk,
                               preferred_element_type=f32) * M
                e = (full[:, 0:_EMB] + full[:, _EMB:2 * _EMB]
                     + full[:, 2 * _EMB:3 * _EMB])
                agg = jax.lax.dot_general(
                    S, e, (((0,), (0,)), ((), ())),
                    preferred_element_type=f32)
                root = jnp.dot(x[k], w_root, preferred_element_type=f32)
                x[k] = jnp.maximum(agg + root + b, 0.0)

    for k in range(_GPB):
        pooled = jnp.max(x[k], axis=0, keepdims=True)
        h = jnp.maximum(jnp.dot(pooled, Wd_ref[...],
                                preferred_element_type=f32) + bd_ref[...],
                        0.0)
        val = jnp.sum(h * Wb_ref[...], axis=1, keepdims=True) + bb_ref[...]
        out_ref[k, :, :] = jnp.broadcast_to(val, (1, 128))


def kernel(unary_tensor, binary_tensor, W_embed, b_embed, W_root, W_rel,
           b_gnn, W_d, b_d, W_b, b_b):
    blk = binary_tensor.reshape(_G * _OBJ, _OBJ * _R)
    unary = unary_tensor.astype(jnp.float32).reshape(_G * _OBJ, _FEAT)
    wstk = jnp.swapaxes(W_rel, 1, 2).reshape(_NBL, _EMB, _RE)
    be = b_embed.reshape(1, _EMB)
    bd = b_d.reshape(1, 128)
    wb = W_b.reshape(1, 128)
    bb = b_b.reshape(1, 1)

    full = lambda *shape: pl.BlockSpec(shape, lambda g: (0,) * len(shape))
    steps = _G // _GPB
    out = pl.pallas_call(
        _gnn_body,
        grid=(steps,),
        in_specs=[
            pl.BlockSpec((_GPB * _OBJ, _RO), lambda g: (g, 0)),
            pl.BlockSpec((_GPB * _OBJ, _FEAT), lambda g: (g, 0)),
            full(_FEAT, _EMB),
            full(1, _EMB),
            full(_NBL, _EMB, _EMB),
            full(_NBL, _EMB, _RE),
            full(_NBL, _EMB),
            full(_EMB, 128),
            full(1, 128),
            full(1, 128),
            full(1, 1),
        ],
        out_specs=pl.BlockSpec((_GPB, 1, 128), lambda g: (g, 0, 0)),
        out_shape=jax.ShapeDtypeStruct((_G, 1, 128), jnp.float32),
        compiler_params=pltpu.CompilerParams(
            dimension_semantics=("parallel",)),
    )(blk, unary, W_embed, be, W_root, wstk, b_gnn, W_d, bd, wb, bb)
    return out[:, 0, 0].reshape(_T, _B)


# speedup vs baseline: 2.3351x; 2.3351x over previous
import jax
import jax.numpy as jnp
from jax.experimental import pallas as pl
from jax.experimental.pallas import tpu as pltpu

_T, _B, _OBJ, _FEAT, _R, _EMB, _NBL, _MP = 2, 8, 128, 64, 3, 16, 2, 2
_G = _T * _B
_GPB = 4
_NPB = _GPB * _OBJ


def _gnn_body(adj_ref, unary_ref, We_ref, be_ref, Wr_ref, Wrel_ref, bg_ref,
              Wd_ref, bd_ref, Wb_ref, bb_ref, out_ref):
    f32 = jnp.float32

    an = []
    for k in range(_GPB):
        blocks = []
        for r in range(_R):
            a = (adj_ref[k, r] != 0).astype(f32)
            deg = jnp.sum(a, axis=0, keepdims=True)
            blocks.append(a * (1.0 / jnp.maximum(deg, 1.0)))
        an.append(jnp.concatenate(blocks, axis=0))

    x = jnp.dot(unary_ref[...], We_ref[...],
                preferred_element_type=f32) + be_ref[...]

    for l in range(_NBL):
        w_root = Wr_ref[l]
        b = bg_ref[l:l + 1, :]
        for _ in range(_MP):
            t = [jnp.dot(x, Wrel_ref[l, r], preferred_element_type=f32)
                 for r in range(_R)]
            root = jnp.dot(x, w_root, preferred_element_type=f32)
            aggs = []
            for k in range(_GPB):
                sl = slice(k * _OBJ, (k + 1) * _OBJ)
                tk = jnp.concatenate([t[r][sl] for r in range(_R)],
                                     axis=0)
                aggs.append(jax.lax.dot_general(
                    an[k], tk, (((0,), (0,)), ((), ())),
                    preferred_element_type=f32))
            x = jnp.maximum(jnp.concatenate(aggs, axis=0) + root + b, 0.0)

    pooled = jnp.concatenate(
        [jnp.max(x[k * _OBJ:(k + 1) * _OBJ], axis=0, keepdims=True)
         for k in range(_GPB)], axis=0)
    h = jnp.maximum(jnp.dot(pooled, Wd_ref[...],
                            preferred_element_type=f32) + bd_ref[...], 0.0)
    val = jnp.sum(h * Wb_ref[...], axis=1, keepdims=True) + bb_ref[...]
    out_ref[...] = jnp.broadcast_to(val[:, :, None], (_GPB, 1, 128))


def kernel(unary_tensor, binary_tensor, W_embed, b_embed, W_root, W_rel,
           b_gnn, W_d, b_d, W_b, b_b):
    adj = binary_tensor.reshape(_G, _OBJ, _OBJ, _R).transpose(0, 3, 1, 2)
    unary = unary_tensor.astype(jnp.float32).reshape(_G * _OBJ, _FEAT)
    be = b_embed.reshape(1, _EMB)
    bd = b_d.reshape(1, 128)
    wb = W_b.reshape(1, 128)
    bb = b_b.reshape(1, 1)

    full = lambda *shape: pl.BlockSpec(shape, lambda g: (0,) * len(shape))
    out = pl.pallas_call(
        _gnn_body,
        grid=(_G // _GPB,),
        in_specs=[
            pl.BlockSpec((_GPB, _R, _OBJ, _OBJ), lambda g: (g, 0, 0, 0)),
            pl.BlockSpec((_NPB, _FEAT), lambda g: (g, 0)),
            full(_FEAT, _EMB),
            full(1, _EMB),
            full(_NBL, _EMB, _EMB),
            full(_NBL, _R, _EMB, _EMB),
            full(_NBL, _EMB),
            full(_EMB, 128),
            full(1, 128),
            full(1, 128),
            full(1, 1),
        ],
        out_specs=pl.BlockSpec((_GPB, 1, 128), lambda g: (g, 0, 0)),
        out_shape=jax.ShapeDtypeStruct((_G, 1, 128), jnp.float32),
        compiler_params=pltpu.CompilerParams(
            dimension_semantics=("parallel",)),
    )(adj, unary, W_embed, be, W_root, W_rel, b_gnn, W_d, bd, wb, bb)
    return out[:, 0, 0].reshape(_T, _B)


# speedup vs baseline: 2.8356x; 1.2144x over previous
import jax
import jax.numpy as jnp
from jax.experimental import pallas as pl
from jax.experimental.pallas import tpu as pltpu

_T, _B, _OBJ, _FEAT, _R, _EMB, _NBL, _MP = 2, 8, 128, 64, 3, 16, 2, 2
_G = _T * _B
_GPB = 8
_NPB = _GPB * _OBJ


def _gnn_body(adj_ref, unary_ref, We_ref, be_ref, Wr_ref, Wrel_ref, bg_ref,
              Wd_ref, bd_ref, Wb_ref, bb_ref, out_ref):
    f32 = jnp.float32

    an = []
    for k in range(_GPB):
        blocks = []
        for r in range(_R):
            a = (adj_ref[k, r] != 0).astype(f32)
            deg = jnp.sum(a, axis=0, keepdims=True)
            blocks.append(a * (1.0 / jnp.maximum(deg, 1.0)))
        an.append(jnp.concatenate(blocks, axis=0))

    x = jnp.dot(unary_ref[...].reshape(_NPB, _FEAT), We_ref[...],
                preferred_element_type=f32) + be_ref[...]

    for l in range(_NBL):
        w_root = Wr_ref[l]
        b = bg_ref[l:l + 1, :]
        for _ in range(_MP):
            t = [jnp.dot(x, Wrel_ref[l, r], preferred_element_type=f32)
                 for r in range(_R)]
            root = jnp.dot(x, w_root, preferred_element_type=f32)
            aggs = []
            for k in range(_GPB):
                sl = slice(k * _OBJ, (k + 1) * _OBJ)
                tk = jnp.concatenate([t[r][sl] for r in range(_R)],
                                     axis=0)
                aggs.append(jax.lax.dot_general(
                    an[k], tk, (((0,), (0,)), ((), ())),
                    preferred_element_type=f32))
            x = jnp.maximum(jnp.concatenate(aggs, axis=0) + root + b, 0.0)

    pooled = jnp.concatenate(
        [jnp.max(x[k * _OBJ:(k + 1) * _OBJ], axis=0, keepdims=True)
         for k in range(_GPB)], axis=0)
    h = jnp.maximum(jnp.dot(pooled, Wd_ref[...],
                            preferred_element_type=f32) + bd_ref[...], 0.0)
    val = jnp.dot(h, Wb_ref[...],
                  preferred_element_type=f32) + bb_ref[...]

    gi = jax.lax.broadcasted_iota(jnp.int32, (_GPB, _GPB), 0)
    bi = jax.lax.broadcasted_iota(jnp.int32, (_GPB, _GPB), 1)
    row = jnp.sum(jnp.where(gi == bi, val, 0.0), axis=0,
                  keepdims=True)
    j = pl.program_id(0)
    out_ref[pl.ds(j, 1), :] = row


def kernel(unary_tensor, binary_tensor, W_embed, b_embed, W_root, W_rel,
           b_gnn, W_d, b_d, W_b, b_b):
    adj = binary_tensor.reshape(_G, _OBJ, _OBJ, _R).transpose(0, 3, 1, 2)
    unary = unary_tensor.astype(jnp.float32).reshape(_G * _OBJ, _FEAT)
    be = b_embed.reshape(1, _EMB)
    bd = b_d.reshape(1, 128)
    bb = b_b.reshape(1, 1)

    full = lambda *shape: pl.BlockSpec(shape, lambda g: (0,) * len(shape))
    return pl.pallas_call(
        _gnn_body,
        grid=(_G // _GPB,),
        in_specs=[
            pl.BlockSpec((_GPB, _R, _OBJ, _OBJ), lambda g: (g, 0, 0, 0)),
            pl.BlockSpec((_NPB, _FEAT), lambda g: (g, 0)),
            full(_FEAT, _EMB),
            full(1, _EMB),
            full(_NBL, _EMB, _EMB),
            full(_NBL, _R, _EMB, _EMB),
            full(_NBL, _EMB),
            full(_EMB, 128),
            full(1, 128),
            full(128, 1),
            full(1, 1),
        ],
        out_specs=pl.BlockSpec((_T, _B), lambda g: (0, 0)),
        out_shape=jax.ShapeDtypeStruct((_T, _B), jnp.float32),
        compiler_params=pltpu.CompilerParams(
            dimension_semantics=("arbitrary",)),
    )(adj, unary, W_embed, be, W_root, W_rel, b_gnn, W_d, bd, W_b, bb)


# speedup vs baseline: 4.6069x; 1.6246x over previous
import jax
import jax.numpy as jnp
from jax.experimental import pallas as pl
from jax.experimental.pallas import tpu as pltpu

_T, _B, _OBJ, _FEAT, _R, _EMB, _NBL, _MP = 2, 8, 128, 64, 3, 16, 2, 2
_G = _T * _B
_GPB = 8
_NPB = _GPB * _OBJ


def _gnn_body(adj_ref, unary_ref, We_ref, be_ref, Wr_ref, Wrel_ref, bg_ref,
              Wd_ref, bd_ref, Wb_ref, bb_ref, out_ref):
    f32 = jnp.float32

    an = []
    for k in range(_GPB):
        blocks = []
        for r in range(_R):
            a = (adj_ref[k, r] != 0).astype(f32)
            deg = jnp.sum(a, axis=0, keepdims=True)
            blocks.append(a * (1.0 / jnp.maximum(deg, 1.0)))
        an.append(jnp.concatenate(blocks, axis=0))

    x = jnp.concatenate(
        [jax.lax.dot_general(unary_ref[k], We_ref[...],
                             (((0,), (1,)), ((), ())),
                             preferred_element_type=f32)
         for k in range(_GPB)], axis=0) + be_ref[...]

    for l in range(_NBL):
        w_root = Wr_ref[l]
        b = bg_ref[l:l + 1, :]
        for _ in range(_MP):
            t = [jnp.dot(x, Wrel_ref[l, r], preferred_element_type=f32)
                 for r in range(_R)]
            root = jnp.dot(x, w_root, preferred_element_type=f32)
            aggs = []
            for k in range(_GPB):
                sl = slice(k * _OBJ, (k + 1) * _OBJ)
                tk = jnp.concatenate([t[r][sl] for r in range(_R)],
                                     axis=0)
                aggs.append(jax.lax.dot_general(
                    an[k], tk, (((0,), (0,)), ((), ())),
                    preferred_element_type=f32))
            x = jnp.maximum(jnp.concatenate(aggs, axis=0) + root + b, 0.0)

    pooled = jnp.concatenate(
        [jnp.max(x[k * _OBJ:(k + 1) * _OBJ], axis=0, keepdims=True)
         for k in range(_GPB)], axis=0)
    h = jnp.maximum(jnp.dot(pooled, Wd_ref[...],
                            preferred_element_type=f32) + bd_ref[...], 0.0)
    val = jnp.sum(h * Wb_ref[...], axis=1,
                  keepdims=True) + bb_ref[...]

    gi = jax.lax.broadcasted_iota(jnp.int32, (_GPB, _GPB), 0)
    bi = jax.lax.broadcasted_iota(jnp.int32, (_GPB, _GPB), 1)
    row = jnp.sum(jnp.where(gi == bi, val, 0.0), axis=0,
                  keepdims=True)
    j = pl.program_id(0)
    out_ref[pl.ds(j, 1), :] = row


def kernel(unary_tensor, binary_tensor, W_embed, b_embed, W_root, W_rel,
           b_gnn, W_d, b_d, W_b, b_b):
    adj = binary_tensor.reshape(_G, _OBJ, _OBJ, _R).transpose(0, 3, 1, 2)
    unary = jnp.swapaxes(unary_tensor.astype(jnp.float32), 2, 3).reshape(
        _G, _FEAT, _OBJ)
    wet = W_embed.T
    wb = W_b.reshape(1, 128)
    be = b_embed.reshape(1, _EMB)
    bd = b_d.reshape(1, 128)
    bb = b_b.reshape(1, 1)

    full = lambda *shape: pl.BlockSpec(shape, lambda g: (0,) * len(shape))
    return pl.pallas_call(
        _gnn_body,
        grid=(_G // _GPB,),
        in_specs=[
            pl.BlockSpec((_GPB, _R, _OBJ, _OBJ), lambda g: (g, 0, 0, 0)),
            pl.BlockSpec((_GPB, _FEAT, _OBJ), lambda g: (g, 0, 0)),
            full(_EMB, _FEAT),
            full(1, _EMB),
            full(_NBL, _EMB, _EMB),
            full(_NBL, _R, _EMB, _EMB),
            full(_NBL, _EMB),
            full(_EMB, 128),
            full(1, 128),
            full(1, 128),
            full(1, 1),
        ],
        out_specs=pl.BlockSpec((_T, _B), lambda g: (0, 0)),
        out_shape=jax.ShapeDtypeStruct((_T, _B), jnp.float32),
        compiler_params=pltpu.CompilerParams(
            dimension_semantics=("arbitrary",)),
    )(adj, unary, wet, be, W_root, W_rel, b_gnn, W_d, bd, wb, bb)
